# baseline (speedup 1.0000x reference)
import jax
import jax.numpy as jnp
from jax.experimental import pallas as pl

N_ENT = 50000
EDGE_T = 16
NUM_SAMPLE = 4
ITEM_LO = 0
ITEM_HI = 24999
K_STEP = 2


def _sage(x, src, dst, Wl, bl, Wr, br):
    n = x.shape[0]
    agg = jax.ops.segment_sum(x[src], dst, num_segments=n)
    cnt = jax.ops.segment_sum(jnp.ones((src.shape[0],), x.dtype), dst, num_segments=n)
    agg = agg / jnp.clip(cnt, 1.0, None)[:, None]
    return agg @ Wl.T + bl + x @ Wr.T + br


def _gcn(x, src, dst, W1l, b1l, W1r, b1r, W2l, b2l, W2r, b2r):
    h = _sage(x, src, dst, W1l, b1l, W1r, b1r)
    h = jnp.where(h >= 0, h, 0.01 * h)
    h = _sage(h, src, dst, W2l, b2l, W2r, b2r)
    nrm = jnp.sqrt(jnp.sum(h * h, axis=1, keepdims=True))
    return h / jnp.maximum(nrm, 1e-12)


def _kg_step(emb, pos, user, adj_matrix, step):
    u_e = emb[user]
    pos_e = emb[pos][:, None, :]
    one_hop = adj_matrix[pos]
    i_e = emb[one_hop]
    p = jnp.einsum('btd,bd->bt', pos_e * i_e, u_e)
    logits = jax.nn.softmax(p, axis=1)
    if step == 1:
        nid = jnp.argmax(logits, axis=1)[:, None]
    else:
        nid = jnp.argsort(logits, axis=1)[:, :NUM_SAMPLE]
    cand = jnp.take_along_axis(one_hop, nid, axis=1)
    clog = jnp.log(jnp.take_along_axis(logits, nid, axis=1))
    if step == 1:
        cand = cand[:, 0]
        clog = clog[:, 0]
    return cand, clog


def _filter_entity(neg, key):
    rnd = jax.random.randint(key, neg.shape, ITEM_LO, ITEM_HI + 1)
    neg = jnp.where(neg > ITEM_HI, rnd, neg)
    neg = jnp.where(neg < ITEM_LO, rnd, neg)
    return neg


def _dis_step(dis_user_emb, dis_item_emb, negs, users, logits):
    u = dis_user_emb[users]
    i = dis_item_emb[negs]
    ranking = jnp.sum(u[:, None, :] * i, axis=-1)
    idx = jnp.argmax(ranking, axis=1)[:, None]
    good_neg = jnp.take_along_axis(negs, idx, axis=1)[:, 0]
    good_logits = jnp.take_along_axis(logits, idx, axis=1)[:, 0]
    return good_neg, good_logits


def _filter_trainset(negs, train_set, random_set):
    in_train = jnp.sum(negs[:, None] == train_set, axis=1)
    return jnp.where(in_train > 0, random_set, negs)


def _identity_pallas(a, b):
    def body(a_ref, b_ref, oa_ref, ob_ref):
        oa_ref[...] = a_ref[...]
        ob_ref[...] = b_ref[...]
    return pl.pallas_call(
        body,
        out_shape=(jax.ShapeDtypeStruct(a.shape, a.dtype),
                   jax.ShapeDtypeStruct(b.shape, b.dtype)),
    )(a, b)


def kernel(entity_embedding, W1l, b1l, W1r, b1r, W2l, b2l, W2r, b2r,
           dis_user_emb, dis_item_emb, u_id, pos_i_id, neg_i_id,
           adj_matrix, edge_matrix, train_set):
    n_node = edge_matrix.shape[0]
    src = jnp.repeat(jnp.arange(n_node), EDGE_T)
    dst = edge_matrix.reshape(-1)
    emb = _gcn(entity_embedding, src, dst, W1l, b1l, W1r, b1r, W2l, b2l, W2r, b2r)
    rkey = jax.random.key(42)
    pos_cur = pos_i_id
    neg_cols = []
    log_cols = []
    for s in range(K_STEP):
        one_hop_sel, lg1 = _kg_step(emb, pos_cur, u_id, adj_matrix, 1)
        cand, lg2 = _kg_step(emb, one_hop_sel, u_id, adj_matrix, 2)
        rkey, sk = jax.random.split(rkey)
        cand = _filter_entity(cand, sk)
        gneg, glg = _dis_step(dis_user_emb, dis_item_emb, cand, u_id, lg2)
        gneg = _filter_trainset(gneg, train_set, neg_i_id)
        neg_cols.append(gneg[:, None])
        log_cols.append((lg1 + glg)[:, None])
        pos_cur = gneg
    negs = jnp.concatenate(neg_cols, axis=-1)
    logs = jnp.concatenate(log_cols, axis=-1)
    return _identity_pallas(negs, logs)


# speedup vs baseline: 7.1172x; 7.1172x over previous
import functools

import jax
import jax.numpy as jnp
from jax import lax
from jax.experimental import pallas as pl
from jax.experimental.pallas import tpu as pltpu
from jax.experimental.pallas import tpu_sc as plsc

N_ENT = 50000
N_PAD = 50176
EDGE_T = 16
D_IN = 64
D_HID = 32
D_HALF = 16
NUM_SAMPLE = 4
ITEM_HI = 24999
BATCH = 1024
K_STEP = 2

N_EDGE = N_PAD * EDGE_T
EPW = N_EDGE // 32
CHUNK = 128
IDXB = 1792
NBLK = EPW // IDXB
CPB = IDXB // CHUNK
NPC = CHUNK // EDGE_T
RPB = IDXB // EDGE_T
RPT = N_PAD // 16
SUBR = 112
NSUB = RPT // SUBR
ROWS_W = BATCH // 32

_MESH = plsc.VectorSubcoreMesh(core_axis_name="c", subcore_axis_name="s")
_SC_PARAMS = pltpu.CompilerParams(needs_layout_passes=False,
                                  use_tc_tiling_on_sc=False)


def _lane0():
    return lax.iota(jnp.int32, 16) == 0


def _sstore1(ref, i, val):
    plsc.store_scatter(ref, (jnp.full((16,), i, jnp.int32),),
                       jnp.full((16,), val, ref.dtype), mask=_lane0())


def _sstore2(ref, r, c, val):
    plsc.store_scatter(ref, (jnp.full((16,), r, jnp.int32),
                             jnp.full((16,), c, jnp.int32)),
                       jnp.full((16,), val, ref.dtype), mask=_lane0())


def _extract(v, i):
    lanes = lax.iota(jnp.int32, 16)
    return jnp.sum(jnp.where(lanes == i, v, jnp.zeros_like(v)))


def _bf16r(v):
    u = plsc.bitcast(v, jnp.uint32)
    r = (u + jnp.uint32(0x7FFF) + ((u >> jnp.uint32(16)) & jnp.uint32(1))) \
        & jnp.uint32(0xFFFF0000)
    return plsc.bitcast(r, jnp.float32)


_R = 3584


def _tc_layer1(acc, cnt2d, x_pad, w1lT, w1rT, b1l, b1r):
    def body(acc_ref, cnt_ref, x_ref, wl_ref, wr_ref, bl_ref, br_ref,
             lo_ref, hi_ref):
        i = pl.program_id(0)
        a = jnp.concatenate([acc_ref[q, 0] + acc_ref[q, 1] for q in range(4)],
                            axis=1)
        agg = a / jnp.clip(cnt_ref[...], 1.0, None)
        h = (jnp.dot(agg, wl_ref[...], preferred_element_type=jnp.float32)
             + bl_ref[...]
             + jnp.dot(x_ref[...], wr_ref[...], preferred_element_type=jnp.float32)
             + br_ref[...])
        h = jnp.where(h >= 0, h, 0.01 * h)
        rows = lax.broadcasted_iota(jnp.int32, (_R, D_HID), 0) + i * _R
        h = jnp.where(rows < N_ENT, h, 0.0)
        lo_ref[...] = h[:, :D_HALF]
        hi_ref[...] = h[:, D_HALF:]
    return pl.pallas_call(
        body,
        grid=(N_PAD // _R,),
        in_specs=[pl.BlockSpec((4, 2, _R, D_HALF), lambda i: (0, 0, i, 0)),
                  pl.BlockSpec((_R, 1), lambda i: (i, 0)),
                  pl.BlockSpec((_R, D_IN), lambda i: (i, 0)),
                  pl.BlockSpec((D_IN, D_HID), lambda i: (0, 0)),
                  pl.BlockSpec((D_IN, D_HID), lambda i: (0, 0)),
                  pl.BlockSpec((1, D_HID), lambda i: (0, 0)),
                  pl.BlockSpec((1, D_HID), lambda i: (0, 0))],
        out_specs=(pl.BlockSpec((_R, D_HALF), lambda i: (i, 0)),
                   pl.BlockSpec((_R, D_HALF), lambda i: (i, 0))),
        out_shape=(jax.ShapeDtypeStruct((N_PAD, D_HALF), jnp.float32),
                   jax.ShapeDtypeStruct((N_PAD, D_HALF), jnp.float32)),
    )(acc, cnt2d, x_pad, w1lT, w1rT, b1l, b1r)


def _tc_layer2(acc2, cnt2d, h1_lo, h1_hi, w2lT, b2l, w2rT, b2r):
    def body(acc_ref, cnt_ref, lo_ref, hi_ref, wl_ref, bl_ref, wr_ref, br_ref,
             o_ref):
        a = jnp.concatenate([acc_ref[0, 0] + acc_ref[0, 1],
                             acc_ref[1, 0] + acc_ref[1, 1]], axis=1)
        agg = a / jnp.clip(cnt_ref[...], 1.0, None)
        h1 = jnp.concatenate([lo_ref[...], hi_ref[...]], axis=1)
        t = (jnp.dot(agg, wl_ref[...], preferred_element_type=jnp.float32)
             + bl_ref[...]
             + jnp.dot(h1, wr_ref[...], preferred_element_type=jnp.float32)
             + br_ref[...])
        nrm = jnp.sqrt(jnp.sum(t * t, axis=1, keepdims=True))
        t = t / jnp.maximum(nrm, 1e-12)
        o_ref[...] = jnp.concatenate([t, jnp.zeros((_R, 128 - D_IN),
                                                   jnp.float32)], axis=1)
    return pl.pallas_call(
        body,
        grid=(N_PAD // _R,),
        in_specs=[pl.BlockSpec((2, 2, _R, D_HALF), lambda i: (0, 0, i, 0)),
                  pl.BlockSpec((_R, 1), lambda i: (i, 0)),
                  pl.BlockSpec((_R, D_HALF), lambda i: (i, 0)),
                  pl.BlockSpec((_R, D_HALF), lambda i: (i, 0)),
                  pl.BlockSpec((D_HID, D_IN), lambda i: (0, 0)),
                  pl.BlockSpec((1, D_IN), lambda i: (0, 0)),
                  pl.BlockSpec((D_HID, D_IN), lambda i: (0, 0)),
                  pl.BlockSpec((1, D_IN), lambda i: (0, 0))],
        out_specs=pl.BlockSpec((_R, 128), lambda i: (i, 0)),
        out_shape=jax.ShapeDtypeStruct((N_PAD, 128), jnp.float32),
    )(acc2, cnt2d, h1_lo, h1_hi, w2lT, b2l, w2rT, b2r)


def _tc_finalize(negs_pad, aux):
    def body(negs_ref, aux_ref, on_ref, ol_ref):
        on_ref[...] = negs_ref[:, :K_STEP]
        a = aux_ref[...]
        cols = []
        for s in range(K_STEP):
            z1 = a[:, 4 * s + 0:4 * s + 1]
            d2 = a[:, 4 * s + 1:4 * s + 2]
            z2 = a[:, 4 * s + 2:4 * s + 3]
            cols.append(d2 - jnp.log(z1 * z2))
        ol_ref[...] = jnp.concatenate(cols, axis=1)
    return pl.pallas_call(
        body,
        out_shape=(jax.ShapeDtypeStruct((BATCH, K_STEP), jnp.int32),
                   jax.ShapeDtypeStruct((BATCH, K_STEP), jnp.float32)),
    )(negs_pad, aux)


def _sc_scatter(vals_parts, dst3d, zacc, zcnt):
    nq = len(vals_parts)

    @functools.partial(
        pl.kernel,
        out_type=(jax.ShapeDtypeStruct((nq, 2, N_PAD, D_HALF), jnp.float32),
                  jax.ShapeDtypeStruct((2 * N_PAD,), jnp.float32)),
        mesh=_MESH,
        compiler_params=_SC_PARAMS,
        scratch_types=[
            pltpu.VMEM_SHARED((N_PAD, D_HALF), jnp.float32),
            pltpu.VMEM_SHARED((N_PAD,), jnp.float32),
            pltpu.VMEM((CPB, CHUNK), jnp.int32),
            pltpu.VMEM((RPB, D_HALF), jnp.float32),
            pltpu.VMEM((CHUNK, D_HALF), jnp.float32),
            pltpu.VMEM((CHUNK,), jnp.float32),
            pltpu.VMEM((SUBR, D_HALF), jnp.float32),
            pltpu.VMEM((RPT,), jnp.float32),
        ],
    )
    def k(*args):
        parts_h = args[:nq]
        dst_h, zacc_h, zcnt_h, acc_out, cnt_out = args[nq:nq + 5]
        (acc_sh, cnt_sh, dsti, srows, rows_v, ones_v, bounce,
         cbounce) = args[nq + 5:]
        c = lax.axis_index("c")
        s = lax.axis_index("s")
        w = c * 16 + s
        for j in range(CHUNK // 16):
            ones_v[pl.ds(j * 16, 16)] = jnp.full((16,), 1.0, jnp.float32)

        for half in range(nq):
            vals_h = parts_h[half]

            pltpu.sync_copy(zacc_h, bounce)

            def zero(sub, carry):
                pltpu.sync_copy(bounce,
                                acc_sh.at[pl.ds(s * RPT + sub * SUBR, SUBR)])
                return carry
            lax.fori_loop(0, NSUB, zero, 0)
            if half == 0:
                pltpu.sync_copy(zcnt_h, cbounce)
                pltpu.sync_copy(cbounce, cnt_sh.at[pl.ds(s * RPT, RPT)])
            plsc.subcore_barrier()

            def blk(b, carry):
                pltpu.sync_copy(dst_h.at[w * NBLK + b], dsti)
                pltpu.sync_copy(vals_h.at[pl.ds(w * (EPW // EDGE_T) + b * RPB,
                                                RPB)], srows)

                def chunk(j, carry2):
                    for r in range(NPC):
                        v = srows[j * NPC + r, pl.ds(0, D_HALF)]
                        for q in range(EDGE_T):
                            rows_v[r * EDGE_T + q, pl.ds(0, D_HALF)] = v
                    di = dsti.at[j]
                    pltpu.sync_copy(rows_v, acc_sh.at[di], add=True)
                    if half == 0:
                        pltpu.sync_copy(ones_v, cnt_sh.at[di], add=True)
                    return carry2

                lax.fori_loop(0, CPB, chunk, 0)
                return carry

            lax.fori_loop(0, NBLK, blk, 0)
            plsc.subcore_barrier()

            def wback(sub, carry):
                r0 = s * RPT + sub * SUBR
                pltpu.sync_copy(acc_sh.at[pl.ds(r0, SUBR)], bounce)
                pltpu.sync_copy(bounce, acc_out.at[half, c, pl.ds(r0, SUBR)])
                return carry
            lax.fori_loop(0, NSUB, wback, 0)
            if half == 0:
                pltpu.sync_copy(cnt_sh.at[pl.ds(s * RPT, RPT)], cbounce)
                pltpu.sync_copy(cbounce,
                                cnt_out.at[pl.ds(c * N_PAD + s * RPT, RPT)])

    return k(*vals_parts, dst3d, zacc, zcnt)


def _sc_sample(emb, adj, dis_u, dis_i, u_id, pos0, negrand, train_pad, rnd_all):
    @functools.partial(
        pl.kernel,
        out_type=(jax.ShapeDtypeStruct((BATCH, 16), jnp.int32),
                  jax.ShapeDtypeStruct((BATCH, 16), jnp.float32)),
        mesh=_MESH,
        compiler_params=_SC_PARAMS,
        scratch_types=[
            pltpu.VMEM((ROWS_W,), jnp.int32),
            pltpu.VMEM((ROWS_W,), jnp.int32),
            pltpu.VMEM((ROWS_W,), jnp.int32),
            pltpu.VMEM((ROWS_W + 16,), jnp.int32),
            pltpu.VMEM((ROWS_W, 128), jnp.float32),
            pltpu.VMEM((ROWS_W, 128), jnp.float32),
            pltpu.VMEM((ROWS_W, 128), jnp.float32),
            pltpu.VMEM((ROWS_W, 128), jnp.int32),
            pltpu.VMEM((ROWS_W * EDGE_T, 128), jnp.float32),
            pltpu.VMEM((ROWS_W * NUM_SAMPLE + 16,), jnp.int32),
            pltpu.VMEM((ROWS_W * NUM_SAMPLE + 16,), jnp.float32),
            pltpu.VMEM((ROWS_W + 16,), jnp.float32),
            pltpu.VMEM((ROWS_W * NUM_SAMPLE, 128), jnp.float32),
            pltpu.VMEM((ROWS_W, 64), jnp.int32),
            pltpu.VMEM((ROWS_W * NUM_SAMPLE + 32,), jnp.int32),
            pltpu.VMEM((ROWS_W, 16), jnp.int32),
            pltpu.VMEM((ROWS_W, 16), jnp.float32),
            pltpu.SemaphoreType.DMA,
        ],
    )
    def k(emb_h, adj_h, disu_h, disi_h, uid_h, pos0_h, negrand_h, train_h,
          rnd_h, negs_out, aux_out,
          uids, posids, selids, negrand_v, ue, du, pose, onehop, ie,
          cand_ids, d2_buf, z2_buf, disi, train_v, rnd_v, negs_buf,
          aux_buf, gsem):
        c = lax.axis_index("c")
        s = lax.axis_index("s")
        w = c * 16 + s
        base = w * ROWS_W

        pltpu.sync_copy(uid_h.at[pl.ds(base, ROWS_W)], uids)
        pltpu.sync_copy(pos0_h.at[pl.ds(base, ROWS_W)], posids)
        pltpu.sync_copy(negrand_h.at[pl.ds(base, ROWS_W)],
                        negrand_v.at[pl.ds(0, ROWS_W)])
        pltpu.sync_copy(train_h.at[pl.ds(base, ROWS_W)], train_v)
        pltpu.sync_copy(emb_h.at[uids], ue)
        pltpu.sync_copy(disu_h.at[uids], du)

        def gather_ie(ids_ref):
            def fire(n, carry):
                idxv = ids_ref[n, pl.ds(0, 16)]
                pltpu.async_copy(emb_h.at[idxv],
                                 ie.at[pl.ds(n * EDGE_T, EDGE_T)], gsem)
                return carry
            lax.fori_loop(0, ROWS_W, fire, 0)
            pltpu.make_async_copy(emb_h.at[pl.ds(0, ROWS_W * EDGE_T)],
                                  ie, gsem).wait()

        lanes = lax.iota(jnp.int32, 16)

        def score_row(n):
            pk, uk = [], []
            for kk in range(D_IN // 16):
                pk.append(pose[n, pl.ds(kk * 16, 16)])
                uk.append(_bf16r(ue[n, pl.ds(kk * 16, 16)]))
            pv = jnp.zeros((16,), jnp.float32)
            for t in range(EDGE_T):
                acc = jnp.zeros((16,), jnp.float32)
                for kk in range(D_IN // 16):
                    a = _bf16r(pk[kk] * ie[n * EDGE_T + t, pl.ds(kk * 16, 16)])
                    acc = acc + a * uk[kk]
                pv = jnp.where(lanes == t, jnp.full((16,), jnp.sum(acc),
                                                    jnp.float32), pv)
            return pv

        for step in range(K_STEP):
            pltpu.sync_copy(emb_h.at[posids], pose)
            pltpu.sync_copy(adj_h.at[posids], onehop)
            gather_ie(onehop)

            def loop1(n, carry):
                pv = score_row(n)
                m = jnp.max(pv)
                z = jnp.sum(jnp.exp(pv - m))
                tstar = jnp.max(plsc.all_reduce_ffs(pv == m))
                hop = onehop[n, pl.ds(0, 16)]
                _sstore1(selids, n, _extract(hop, tstar))
                _sstore2(aux_buf, n, 4 * step + 0, z)
                _sstore2(aux_buf, n, 4 * step + 3, 0.0)
                return carry
            lax.fori_loop(0, ROWS_W, loop1, 0)

            pltpu.sync_copy(emb_h.at[selids], pose)
            pltpu.sync_copy(adj_h.at[selids], onehop)
            gather_ie(onehop)
            pltpu.sync_copy(
                rnd_h.at[pl.ds(step * BATCH * NUM_SAMPLE + base * NUM_SAMPLE,
                               ROWS_W * NUM_SAMPLE)],
                rnd_v.at[pl.ds(0, ROWS_W * NUM_SAMPLE)])

            def loop2(n, carry):
                pv = score_row(n)
                m = jnp.max(pv)
                z = jnp.sum(jnp.exp(pv - m))
                _sstore1(z2_buf, n, z)
                ks, vs = plsc.sort_key_val(pv, lanes)
                hop = onehop[n, pl.ds(0, 16)]
                rnd16 = rnd_v[pl.ds(n * NUM_SAMPLE, 16)]
                for j in range(NUM_SAMPLE):
                    tj = vs[j]
                    cid = _extract(hop, tj)
                    rv = rnd16[j]
                    bad = (cid > ITEM_HI) | (cid < 0)
                    cid = jnp.where(bad, rv, cid)
                    _sstore1(cand_ids, n * NUM_SAMPLE + j, cid)
                    _sstore1(d2_buf, n * NUM_SAMPLE + j, ks[j] - m)
                return carry
            lax.fori_loop(0, ROWS_W, loop2, 0)

            pltpu.sync_copy(
                disi_h.at[cand_ids.at[pl.ds(0, ROWS_W * NUM_SAMPLE)]], disi)

            def loop3(n, carry):
                ranks = []
                for j in range(NUM_SAMPLE):
                    acc = (du[n, pl.ds(0, 16)]
                           * disi[n * NUM_SAMPLE + j, pl.ds(0, 16)])
                    for kk in range(1, D_IN // 16):
                        acc = acc + (du[n, pl.ds(kk * 16, 16)]
                                     * disi[n * NUM_SAMPLE + j,
                                            pl.ds(kk * 16, 16)])
                    ranks.append(jnp.sum(acc))
                best = ranks[0]
                bi = jnp.int32(0)
                for j in range(1, NUM_SAMPLE):
                    upd = ranks[j] > best
                    bi = jnp.where(upd, jnp.int32(j), bi)
                    best = jnp.where(upd, ranks[j], best)
                cid16 = cand_ids[pl.ds(n * NUM_SAMPLE, 16)]
                d216 = d2_buf[pl.ds(n * NUM_SAMPLE, 16)]
                gneg = _extract(cid16, bi)
                d2 = _extract(d216, bi)
                g = jnp.full((16,), gneg, jnp.int32)
                hit = (train_v[n, pl.ds(0, 16)] == g)
                for kk in range(1, 4):
                    hit = hit | (train_v[n, pl.ds(kk * 16, 16)] == g)
                nhits = jnp.max(plsc.all_reduce_population_count(hit))
                gneg = jnp.where(nhits > 0, negrand_v[pl.ds(n, 16)][0], gneg)
                _sstore2(negs_buf, n, step, gneg)
                _sstore2(aux_buf, n, 4 * step + 1, d2)
                _sstore2(aux_buf, n, 4 * step + 2, z2_buf[pl.ds(n, 16)][0])
                _sstore1(posids, n, gneg)
                return carry
            lax.fori_loop(0, ROWS_W, loop3, 0)

        pltpu.sync_copy(negs_buf, negs_out.at[pl.ds(base, ROWS_W)])
        pltpu.sync_copy(aux_buf, aux_out.at[pl.ds(base, ROWS_W)])

    return k(emb, adj, dis_u, dis_i, u_id, pos0, negrand, train_pad, rnd_all)


def kernel(entity_embedding, W1l, b1l, W1r, b1r, W2l, b2l, W2r, b2r,
           dis_user_emb, dis_item_emb, u_id, pos_i_id, neg_i_id,
           adj_matrix, edge_matrix, train_set):
    f32 = jnp.float32
    i32 = jnp.int32

    x_pad = jnp.pad(entity_embedding, ((0, N_PAD - N_ENT), (0, 0)))
    pad_block = (N_ENT + (jnp.arange((N_PAD - N_ENT) * EDGE_T, dtype=i32)
                          % (N_PAD - N_ENT))).reshape(N_PAD - N_ENT, EDGE_T)
    edge_pad = jnp.concatenate([edge_matrix.astype(i32), pad_block], axis=0)
    dst3d = edge_pad.reshape(N_EDGE // IDXB, CPB, CHUNK)
    zacc = jnp.zeros((SUBR, D_HALF), f32)
    zcnt = jnp.zeros((RPT,), f32)

    rkey = jax.random.key(42)
    rnds = []
    for _ in range(K_STEP):
        rkey, sk = jax.random.split(rkey)
        rnds.append(jax.random.randint(sk, (BATCH, NUM_SAMPLE), 0, ITEM_HI + 1))
    rnd_all = jnp.concatenate(rnds, axis=0).astype(i32).reshape(-1)
    train_pad = jnp.pad(train_set.astype(i32),
                        ((0, 0), (0, 64 - train_set.shape[1])),
                        constant_values=-1)
    adj_pad = jnp.pad(adj_matrix.astype(i32), ((0, 0), (0, 128 - EDGE_T)))
    disu_pad = jnp.pad(dis_user_emb, ((0, 0), (0, 128 - D_IN)))
    disi_pad = jnp.pad(dis_item_emb, ((0, 0), (0, 128 - D_IN)))

    xq = [x_pad[:, 16 * q:16 * (q + 1)] for q in range(4)]
    acc1, cnt_parts = _sc_scatter(xq, dst3d, zacc, zcnt)
    cnt2d = (cnt_parts[:N_PAD] + cnt_parts[N_PAD:])[:, None]
    h1_lo, h1_hi = _tc_layer1(acc1, cnt2d, x_pad, W1l.T, W1r.T,
                              b1l[None, :], b1r[None, :])
    acc2, _ = _sc_scatter([h1_lo, h1_hi], dst3d, zacc, zcnt)
    emb = _tc_layer2(acc2, cnt2d, h1_lo, h1_hi, W2l.T,
                     b2l[None, :], W2r.T, b2r[None, :])

    if _DEBUG_JAX_SAMPLE:
        emb64 = emb[:N_ENT, :D_IN]
        rkey = jax.random.key(42)
        pos_cur = pos_i_id
        neg_cols, log_cols = [], []
        for _s in range(K_STEP):
            one_hop_sel, lg1 = _dbg_kg_step(emb64, pos_cur, u_id, adj_matrix, 1)
            cand, lg2 = _dbg_kg_step(emb64, one_hop_sel, u_id, adj_matrix, 2)
            rkey, sk = jax.random.split(rkey)
            rnd = jax.random.randint(sk, cand.shape, 0, ITEM_HI + 1)
            cand = jnp.where((cand > ITEM_HI) | (cand < 0), rnd, cand)
            u = dis_user_emb[u_id]
            ie = dis_item_emb[cand]
            ranking = jnp.sum(u[:, None, :] * ie, axis=-1)
            idx = jnp.argmax(ranking, axis=1)[:, None]
            gneg = jnp.take_along_axis(cand, idx, axis=1)[:, 0]
            glg = jnp.take_along_axis(lg2, idx, axis=1)[:, 0]
            in_train = jnp.sum(gneg[:, None] == train_set, axis=1)
            gneg = jnp.where(in_train > 0, neg_i_id, gneg)
            neg_cols.append(gneg[:, None])
            log_cols.append((lg1 + glg)[:, None])
            pos_cur = gneg
        return (jnp.concatenate(neg_cols, axis=-1),
                jnp.concatenate(log_cols, axis=-1))
    negs_pad, aux = _sc_sample(emb, adj_pad, disu_pad, disi_pad,
                               u_id.astype(i32), pos_i_id.astype(i32),
                               neg_i_id.astype(i32), train_pad, rnd_all)
    negs, logits = _tc_finalize(negs_pad, aux)
    return negs, logits


_DEBUG_JAX_SAMPLE = False


def _dbg_kg_step(emb, pos, user, adj_matrix, step):
    u_e = emb[user]
    pos_e = emb[pos][:, None, :]
    one_hop = adj_matrix[pos]
    i_e = emb[one_hop]
    p = jnp.einsum('btd,bd->bt', pos_e * i_e, u_e)
    logits = jax.nn.softmax(p, axis=1)
    if step == 1:
        nid = jnp.argmax(logits, axis=1)[:, None]
    else:
        nid = jnp.argsort(logits, axis=1)[:, :NUM_SAMPLE]
    cand = jnp.take_along_axis(one_hop, nid, axis=1)
    clog = jnp.log(jnp.take_along_axis(logits, nid, axis=1))
    if step == 1:
        cand = cand[:, 0]
        clog = clog[:, 0]
    return cand, clog
